# Initial kernel scaffold; baseline (speedup 1.0000x reference)
import jax
import jax.numpy as jnp
from jax.experimental import pallas as pl


def kernel(X, muK, cK):
    raise NotImplementedError("write your pallas kernel here")


# speedup vs baseline: 12.0645x; 12.0645x over previous
import functools

import jax
import jax.numpy as jnp
from jax.experimental import pallas as pl

_K_PAD = 1024


def _ncm_kernel(x_ref, mu_ref, ck_ref, out_ref, *, k_valid):
    x = x_ref[...]
    mu = mu_ref[...]
    ck = ck_ref[...]

    xx = jnp.sum(x * x, axis=1, keepdims=True)
    mm = jnp.sum(mu * mu, axis=1)[None, :]
    xm = jax.lax.dot_general(
        x, mu, (((1,), (1,)), ((), ())),
        preferred_element_type=jnp.float32,
    )
    scores = 2.0 * xm - xx - mm

    col = jax.lax.broadcasted_iota(jnp.int32, scores.shape, 1)
    valid = col < k_valid
    min_col = jnp.min(
        jnp.where(valid, scores, jnp.inf), axis=1, keepdims=True
    ) - 1.0
    not_visited = (ck == 0.0) & valid
    out_ref[...] = jnp.where(not_visited, min_col, scores)


def kernel(X, muK, cK):
    M, d = X.shape
    K = muK.shape[0]
    mu_p = jnp.pad(muK, ((0, _K_PAD - K), (0, 0)))
    ck_p = jnp.pad(cK, (0, _K_PAD - K), constant_values=1.0).reshape(1, _K_PAD)
    out = pl.pallas_call(
        functools.partial(_ncm_kernel, k_valid=K),
        out_shape=jax.ShapeDtypeStruct((M, _K_PAD), jnp.float32),
    )(X, mu_p, ck_p)
    return out[:, :K]


# speedup vs baseline: 15.8999x; 1.3179x over previous
import jax
import jax.numpy as jnp
from jax.experimental import pallas as pl


def _ncm_kernel(x_ref, mu_ref, ck_ref, out_ref):
    x = x_ref[...]
    mu = mu_ref[...]
    ck = ck_ref[...]

    xx = jnp.sum(x * x, axis=1, keepdims=True)
    mm = jnp.sum(mu * mu, axis=1)[None, :]
    xm = jax.lax.dot_general(
        x, mu, (((1,), (1,)), ((), ())),
        preferred_element_type=jnp.float32,
    )
    scores = 2.0 * xm - xx - mm

    min_col = jnp.min(scores, axis=1, keepdims=True) - 1.0
    out_ref[...] = jnp.where(ck == 0.0, min_col, scores)


def kernel(X, muK, cK):
    M, _ = X.shape
    K = muK.shape[0]
    return pl.pallas_call(
        _ncm_kernel,
        out_shape=jax.ShapeDtypeStruct((M, K), jnp.float32),
    )(X, muK, cK.reshape(1, K))
